# baseline (speedup 1.0000x reference)
import jax
import jax.numpy as jnp
from jax.experimental import pallas as pl
from jax.experimental.pallas import tpu as pltpu

_J = 8
_M = 8
_JR = 28
_P = 56
_N = 8
_T = 2048


def _mix_kernel(x_ref, L_ref, m1_ref, ir_ref, m2_ref, pa_ref, out_ref, w_ref):
    @pl.when(pl.program_id(0) == 0)
    def _build_w():
        m1 = m1_ref[:].reshape(1, _JR)
        ir = ir_ref[:].reshape(1, _JR)
        m2 = m2_ref[:].reshape(_P, 1)
        pa = pa_ref[:].reshape(_P, 1)
        mask = (m2 == m1).astype(jnp.float32)
        oh_a = (pa == jax.lax.broadcasted_iota(jnp.int32, (_P, _J), 1)).astype(
            jnp.float32)
        oh_r = (jax.lax.broadcasted_iota(jnp.int32, (_J, _JR), 0) == ir).astype(
            jnp.float32)
        lg = jnp.dot(jnp.dot(oh_a, L_ref[:, :],
                             preferred_element_type=jnp.float32),
                     oh_r, preferred_element_type=jnp.float32)
        w_ref[:, :] = lg * mask

    w = w_ref[:, :]
    u = pl.program_id(0) % 4
    out_ref[u] = jax.lax.dot_general(
        w, x_ref[0], (((1,), (0,)), ((), ())),
        preferred_element_type=jnp.float32)


def kernel(x, L, mapping1, idx_r, mapping2, sums, p_next_a, p_next_b):
    del sums, p_next_b
    B, n, Jr, T_, two = x.shape
    nt = T_ // 128
    x4 = x.reshape(n, Jr, nt, 128, two).transpose(0, 1, 2, 4, 3).reshape(
        n, Jr, nt * two, 128)
    m1 = mapping1.astype(jnp.int32)
    ir = idx_r.astype(jnp.int32)
    m2 = mapping2.astype(jnp.int32)
    pa = p_next_a.astype(jnp.int32)
    out4 = pl.pallas_call(
        _mix_kernel,
        grid=(n,),
        in_specs=[
            pl.BlockSpec((1, Jr, nt * two, 128), lambda i: (i, 0, 0, 0)),
            pl.BlockSpec((_J, _J), lambda i: (0, 0)),
            pl.BlockSpec((Jr,), lambda i: (0,)),
            pl.BlockSpec((Jr,), lambda i: (0,)),
            pl.BlockSpec((_P,), lambda i: (0,)),
            pl.BlockSpec((_P,), lambda i: (0,)),
        ],
        out_specs=pl.BlockSpec((4, _P, nt * two, 128),
                               lambda i: (i // 4, 0, 0, 0)),
        out_shape=jax.ShapeDtypeStruct((n, _P, nt * two, 128), jnp.float32),
        scratch_shapes=[pltpu.VMEM((_P, _JR), jnp.float32)],
    )(x4, L, m1, ir, m2, pa)
    out = out4.reshape(n, _P, nt, two, 128).transpose(0, 1, 2, 4, 3).reshape(
        B, n, _P, T_, two)
    return out


# speedup vs baseline: 1.3759x; 1.3759x over previous
import jax
import jax.numpy as jnp
from jax.experimental import pallas as pl
from jax.experimental.pallas import tpu as pltpu

_J = 8
_M = 8
_JR = 28
_P = 56
_N = 8
_T = 2048


def _mix_kernel(x_ref, L_ref, m1_ref, ir_ref, m2_ref, pa_ref, out_ref, w_ref):
    @pl.when(pl.program_id(0) == 0)
    def _build_w():
        m1 = m1_ref[:].reshape(1, _JR)
        ir = ir_ref[:].reshape(1, _JR)
        m2 = m2_ref[:].reshape(_P, 1)
        pa = pa_ref[:].reshape(_P, 1)
        mask = (m2 == m1).astype(jnp.float32)
        oh_a = (pa == jax.lax.broadcasted_iota(jnp.int32, (_P, _J), 1)).astype(
            jnp.float32)
        oh_r = (jax.lax.broadcasted_iota(jnp.int32, (_J, _JR), 0) == ir).astype(
            jnp.float32)
        lg = jnp.dot(jnp.dot(oh_a, L_ref[:, :],
                             preferred_element_type=jnp.float32),
                     oh_r, preferred_element_type=jnp.float32)
        w_ref[:, :] = lg * mask

    w = w_ref[:, :]
    for u in range(x_ref.shape[0]):
        out_ref[u] = jax.lax.dot_general(
            w, x_ref[u], (((1,), (0,)), ((), ())),
            preferred_element_type=jnp.float32)


def kernel(x, L, mapping1, idx_r, mapping2, sums, p_next_a, p_next_b):
    del sums, p_next_b
    B, n, Jr, T_, two = x.shape
    nt = T_ // 128
    x4 = x.reshape(n, Jr, nt, 128, two).transpose(0, 1, 2, 4, 3).reshape(
        n, Jr, nt * two, 128)
    m1 = mapping1.astype(jnp.int32)
    ir = idx_r.astype(jnp.int32)
    m2 = mapping2.astype(jnp.int32)
    pa = p_next_a.astype(jnp.int32)
    out4 = pl.pallas_call(
        _mix_kernel,
        grid=(n // 4,),
        in_specs=[
            pl.BlockSpec((4, Jr, nt * two, 128), lambda i: (i, 0, 0, 0)),
            pl.BlockSpec((_J, _J), lambda i: (0, 0)),
            pl.BlockSpec((Jr,), lambda i: (0,)),
            pl.BlockSpec((Jr,), lambda i: (0,)),
            pl.BlockSpec((_P,), lambda i: (0,)),
            pl.BlockSpec((_P,), lambda i: (0,)),
        ],
        out_specs=pl.BlockSpec((4, _P, nt * two, 128), lambda i: (i, 0, 0, 0)),
        out_shape=jax.ShapeDtypeStruct((n, _P, nt * two, 128), jnp.float32),
        scratch_shapes=[pltpu.VMEM((_P, _JR), jnp.float32)],
    )(x4, L, m1, ir, m2, pa)
    out = out4.reshape(n, _P, nt, two, 128).transpose(0, 1, 2, 4, 3).reshape(
        B, n, _P, T_, two)
    return out
